# baseline (speedup 1.0000x reference)
import functools

import jax
import jax.numpy as jnp
from jax import lax
from jax.experimental import pallas as pl
from jax.experimental.pallas import tpu as pltpu

_BF = jnp.bfloat16
_F32 = jnp.float32


def _cell(g, c, H):
    sg = jax.nn.sigmoid(g[:, 0:3 * H])
    gg = jnp.tanh(g[:, 3 * H:4 * H])
    c_n = sg[:, H:2 * H] * c + sg[:, 0:H] * gg
    h_n = sg[:, 2 * H:3 * H] * jnp.tanh(c_n)
    return h_n, c_n


def _fused_kernel(lens_ref, x_ref, mega1, mega2,
                  out_ref, sc0, sc1, sc2, *, T, BH, Hs, Hm, C, mlp_dims):
    scs = (sc0, sc1, sc2)

    lens = lens_ref[...]
    masks = [(lens > float(t)).astype(_F32) for t in range(T)]
    nmasks = [1.0 - mk for mk in masks]

    def cc(w, n, H):
        if H == Hm:
            return w
        return jnp.concatenate([w[..., j * Hm:j * Hm + H] for j in range(n)],
                               axis=-1)

    wm = []
    for m in range(3):
        H = Hs[m]
        wih1 = cc(mega1[m, 0:H, :], 8, H)
        whh1 = cc(jnp.concatenate([mega1[m, Hm:Hm + H, :],
                                   mega1[m, 2 * Hm:2 * Hm + H, :]], axis=0),
                  8, H)
        wih2 = cc(jnp.concatenate([mega1[m, 3 * Hm:3 * Hm + H, :],
                                   mega1[m, 4 * Hm:4 * Hm + H, :]], axis=0),
                  8, H)
        whh2 = cc(jnp.concatenate([mega1[m, 5 * Hm:5 * Hm + H, :],
                                   mega1[m, 6 * Hm:6 * Hm + H, :]], axis=0),
                  8, H)
        b1 = cc(mega1[m, 7 * Hm:7 * Hm + 1, :], 8, H)
        b2 = cc(mega1[m, 7 * Hm + 1:7 * Hm + 2, :], 8, H)
        lng = cc(mega1[m, 7 * Hm + 2:7 * Hm + 3, 0:2 * Hm], 2, H)
        lnb = cc(mega1[m, 7 * Hm + 3:7 * Hm + 4, 0:2 * Hm], 2, H)
        A1, A2, A3 = mlp_dims
        w1 = jnp.concatenate([mega2[m, q * Hm:q * Hm + H, :]
                              for q in range(4)], axis=0)
        r = 4 * Hm
        c1 = mega2[m, r:r + 1, :]
        w2 = mega2[m, r + 1:r + 1 + A1, 0:A2]
        r += 1 + A1
        c2 = mega2[m, r:r + 1, 0:A2]
        w3 = mega2[m, r + 1:r + 1 + A2, 0:A3]
        r += 1 + A2
        c3 = mega2[m, r:r + 1, 0:A3]
        w4 = mega2[m, r + 1:r + 1 + A3, 0:C]
        r += 1 + A3
        c4 = mega2[m, r:r + 1, 0:C]
        wm.append(dict(wih1=wih1, b1=b1, whh1=whh1, lng=lng.astype(_F32),
                       lnb=lnb.astype(_F32), wih2=wih2, b2=b2, whh2=whh2,
                       w1=w1, c1=c1, w2=w2, c2=c2, w3=w3, c3=c3,
                       w4=w4, c4=c4))

    def step(s, gx, whh, st, H, sc):
        hf, cf, hb, cb = st
        t, tb = s, T - 1 - s
        G = 4 * H
        hcat = jnp.concatenate([hf, hb], axis=-1).astype(_BF)
        g_rec = jnp.dot(hcat, whh, preferred_element_type=_F32)
        gf = gx[t * BH:(t + 1) * BH, 0:G] + g_rec[:, 0:G]
        gb = gx[tb * BH:(tb + 1) * BH, G:2 * G] + g_rec[:, G:2 * G]
        hf_n, cf_n = _cell(gf, cf, H)
        hb_n, cb_n = _cell(gb, cb, H)
        if sc is not None:
            sc[t * BH:(t + 1) * BH, 0:H] = masks[t] * hf_n
            sc[tb * BH:(tb + 1) * BH, H:2 * H] = masks[tb] * hb_n
        hf = masks[t] * hf_n + nmasks[t] * hf
        cf = masks[t] * cf_n + nmasks[t] * cf
        hb = masks[tb] * hb_n + nmasks[tb] * hb
        cb = masks[tb] * cb_n + nmasks[tb] * cb
        return hf, cf, hb, cb

    xall = x_ref[...]
    gx1 = []
    off = 0
    for m in range(3):
        H = Hs[m]
        x = xall[:, :, off:off + H].reshape(T * BH, H)
        off += H
        gx1.append(jnp.dot(x, wm[m]["wih1"], preferred_element_type=_F32)
                   + wm[m]["b1"])

    st1 = [tuple(jnp.zeros((BH, Hs[m]), _F32) for _ in range(4))
           for m in range(3)]
    for s in range(T):
        for m in range(3):
            st1[m] = step(s, gx1[m], wm[m]["whh1"], st1[m], Hs[m], scs[m])

    gx2 = []
    for m in range(3):
        h1 = scs[m][...]
        mean = jnp.mean(h1, axis=-1, keepdims=True)
        cen = h1 - mean
        var = jnp.mean(cen * cen, axis=-1, keepdims=True)
        normed = cen * lax.rsqrt(var + 1e-5) * wm[m]["lng"] + wm[m]["lnb"]
        gx2.append(jnp.dot(normed.astype(_BF), wm[m]["wih2"],
                           preferred_element_type=_F32) + wm[m]["b2"])

    st2 = [tuple(jnp.zeros((BH, Hs[m]), _F32) for _ in range(4))
           for m in range(3)]
    for s in range(T):
        for m in range(3):
            st2[m] = step(s, gx2[m], wm[m]["whh2"], st2[m], Hs[m], None)

    acc = jnp.zeros((BH, C), _F32)
    for m in range(3):
        d = wm[m]
        h1f, _, h1b, _ = st1[m]
        h2f, _, h2b, _ = st2[m]
        feats = jnp.concatenate([h1f, h2f, h1b, h2b], axis=-1).astype(_BF)
        h = jnp.maximum(jnp.dot(feats, d["w1"],
                                preferred_element_type=_F32) + d["c1"], 0.0)
        h = jnp.maximum(jnp.dot(h.astype(_BF), d["w2"],
                                preferred_element_type=_F32) + d["c2"], 0.0)
        h = jnp.maximum(jnp.dot(h.astype(_BF), d["w3"],
                                preferred_element_type=_F32) + d["c3"], 0.0)
        acc = acc + jnp.dot(h.astype(_BF), d["w4"],
                            preferred_element_type=_F32) + d["c4"]
    out_ref[...] = acc * (1.0 / 3.0)


def kernel(w00, w01, w02, w03, w04, w05, w06, w07, w08, w09, w10,
           w11, w12, w13, w14, w15, w16,
           embed, sentences, video, acoustic, lengths):
    Hm = w02.shape[1] // 2
    C = w15.shape[2]
    B, T = sentences.shape
    BH = B // 2
    Hs = (embed.shape[1], video.shape[2], acoustic.shape[2])
    W1 = 8 * Hm
    W2 = w09.shape[2]

    def lanepad(w, W):
        return jnp.pad(w, ((0, 0), (0, 0), (0, W - w.shape[2])))

    mega1 = jnp.concatenate(
        [w00, w02, w06, w08, w01, w07,
         lanepad(w03, W1), lanepad(w04, W1)], axis=1).astype(_BF)
    mega2 = jnp.concatenate(
        [w09, w10, lanepad(w11, W2), lanepad(w12, W2),
         lanepad(w13, W2), lanepad(w14, W2),
         lanepad(w15, W2), lanepad(w16, W2)], axis=1).astype(_BF)

    emb = embed[sentences]
    xcat = jnp.concatenate(
        [jnp.transpose(v, (1, 0, 2)) for v in (emb, video, acoustic)],
        axis=2).astype(_BF)

    lens_col = lengths.astype(_F32).reshape(B, 1)

    kfn = functools.partial(_fused_kernel, T=T, BH=BH, Hs=Hs, Hm=Hm, C=C,
                            mlp_dims=(w09.shape[2], w11.shape[2], w13.shape[2]))

    in_specs = [pl.BlockSpec((BH, 1), lambda i: (i, 0)),
                pl.BlockSpec((T, BH, sum(Hs)), lambda i: (0, i, 0)),
                pl.BlockSpec(mega1.shape, lambda i: (0, 0, 0)),
                pl.BlockSpec(mega2.shape, lambda i: (0, 0, 0))]

    return pl.pallas_call(
        kfn,
        out_shape=jax.ShapeDtypeStruct((B, C), _F32),
        grid=(2,),
        in_specs=in_specs,
        out_specs=pl.BlockSpec((BH, C), lambda i: (i, 0)),
        scratch_shapes=[pltpu.VMEM((T * BH, 2 * H), _F32) for H in Hs],
        compiler_params=pltpu.CompilerParams(
            dimension_semantics=("parallel",)),
    )(lens_col, xcat, mega1, mega2)


# speedup vs baseline: 1.5195x; 1.5195x over previous
import functools

import jax
import jax.numpy as jnp
from jax import lax
from jax.experimental import pallas as pl
from jax.experimental.pallas import tpu as pltpu

_BF = jnp.bfloat16
_F32 = jnp.float32


def _fused_kernel(lens_ref, x_ref,
                  w00, w01, w02, w03, w04, w05, w06, w07, w08, w09, w10,
                  w11, w12, w13, w14, w15, w16,
                  out_ref, h1sc, *, T, BH, Hm, C):
    m_id = pl.program_id(1)
    G = 4 * Hm

    lens = lens_ref[...]
    masks = [(lens > float(t)).astype(_F32) for t in range(T)]
    nmasks = [1.0 - mk for mk in masks]

    def cell(g, c):
        sg = jax.nn.sigmoid(g[:, 0:3 * Hm])
        gg = jnp.tanh(g[:, 3 * Hm:4 * Hm])
        c_n = sg[:, Hm:2 * Hm] * c + sg[:, 0:Hm] * gg
        h_n = sg[:, 2 * Hm:3 * Hm] * jnp.tanh(c_n)
        return h_n, c_n

    def step(s, gx, whh, st, collect):
        hf, cf, hb, cb = st
        t, tb = s, T - 1 - s
        hcat = jnp.concatenate([hf, hb], axis=-1).astype(_BF)
        g_rec = jnp.dot(hcat, whh, preferred_element_type=_F32)
        gf = gx[t * BH:(t + 1) * BH, 0:G] + g_rec[:, 0:G]
        gb = gx[tb * BH:(tb + 1) * BH, G:2 * G] + g_rec[:, G:2 * G]
        hf_n, cf_n = cell(gf, cf)
        hb_n, cb_n = cell(gb, cb)
        if collect:
            h1sc[t * BH:(t + 1) * BH, 0:Hm] = masks[t] * hf_n
            h1sc[tb * BH:(tb + 1) * BH, Hm:2 * Hm] = masks[tb] * hb_n
        hf = masks[t] * hf_n + nmasks[t] * hf
        cf = masks[t] * cf_n + nmasks[t] * cf
        hb = masks[tb] * hb_n + nmasks[tb] * hb
        cb = masks[tb] * cb_n + nmasks[tb] * cb
        return hf, cf, hb, cb

    x = x_ref[0].reshape(T * BH, Hm)
    gx1 = jnp.dot(x, w00[0].astype(_BF), preferred_element_type=_F32) + w01[0]
    whh1 = w02[0].astype(_BF)
    st = tuple(jnp.zeros((BH, Hm), _F32) for _ in range(4))
    for s in range(T):
        st = step(s, gx1, whh1, st, True)
    h1f, h1b = st[0], st[2]

    h1 = h1sc[...]
    lnms = w05[0]
    mean = jnp.sum(h1 * lnms, axis=-1, keepdims=True)
    cen = h1 - mean
    var = jnp.sum(cen * cen * lnms, axis=-1, keepdims=True)
    normed = cen * lax.rsqrt(var + 1e-5) * w03[0] + w04[0]

    gx2 = jnp.dot(normed.astype(_BF), w06[0].astype(_BF),
                  preferred_element_type=_F32) + w07[0]
    whh2 = w08[0].astype(_BF)
    st = tuple(jnp.zeros((BH, Hm), _F32) for _ in range(4))
    for s in range(T):
        st = step(s, gx2, whh2, st, False)
    h2f, h2b = st[0], st[2]

    feats = jnp.concatenate([h1f, h2f, h1b, h2b], axis=-1).astype(_BF)
    h = jnp.maximum(jnp.dot(feats, w09[0].astype(_BF),
                            preferred_element_type=_F32) + w10[0], 0.0)
    h = jnp.maximum(jnp.dot(h.astype(_BF), w11[0].astype(_BF),
                            preferred_element_type=_F32) + w12[0], 0.0)
    h = jnp.maximum(jnp.dot(h.astype(_BF), w13[0].astype(_BF),
                            preferred_element_type=_F32) + w14[0], 0.0)
    logits = (jnp.dot(h.astype(_BF), w15[0].astype(_BF),
                      preferred_element_type=_F32) + w16[0]) * (1. / 3.)

    @pl.when(m_id == 0)
    def _():
        out_ref[...] = logits

    @pl.when(m_id != 0)
    def _():
        out_ref[...] += logits


def kernel(w00, w01, w02, w03, w04, w05, w06, w07, w08, w09, w10,
           w11, w12, w13, w14, w15, w16,
           embed, sentences, video, acoustic, lengths):
    Hm = w02.shape[1] // 2
    C = w15.shape[2]
    B, T = sentences.shape
    BH = B // 2

    emb = embed[sentences]

    def prep(v):
        D = v.shape[2]
        x = jnp.transpose(v, (1, 0, 2)).astype(_BF)
        return jnp.pad(x, ((0, 0), (0, 0), (0, Hm - D)))

    x3 = jnp.stack([prep(emb), prep(video), prep(acoustic)])
    lens_col = lengths.astype(_F32).reshape(B, 1)

    weights = (w00, w01, w02, w03, w04, w05, w06, w07, w08, w09, w10,
               w11, w12, w13, w14, w15, w16)

    kfn = functools.partial(_fused_kernel, T=T, BH=BH, Hm=Hm, C=C)

    in_specs = [pl.BlockSpec((BH, 1), lambda i, m: (i, 0)),
                pl.BlockSpec((1, T, BH, Hm), lambda i, m: (m, 0, i, 0))]
    in_specs += [pl.BlockSpec((1,) + w.shape[1:], lambda i, m: (m, 0, 0))
                 for w in weights]

    return pl.pallas_call(
        kfn,
        out_shape=jax.ShapeDtypeStruct((B, C), _F32),
        grid=(2, 3),
        in_specs=in_specs,
        out_specs=pl.BlockSpec((BH, C), lambda i, m: (i, 0)),
        scratch_shapes=[pltpu.VMEM((T * BH, 2 * Hm), _F32)],
        compiler_params=pltpu.CompilerParams(
            dimension_semantics=("parallel", "arbitrary")),
    )(lens_col, x3, *weights)


# speedup vs baseline: 1.7615x; 1.1592x over previous
import functools

import jax
import jax.numpy as jnp
from jax import lax
from jax.experimental import pallas as pl
from jax.experimental.pallas import tpu as pltpu

_BF = jnp.bfloat16
_F32 = jnp.float32


def _fused_kernel(lens_ref, x_ref, wbig,
                  w00, w01, w03, w04, w05, w07, w09, w10,
                  w11, w12, w13, w14, w15, w16,
                  out_ref, h1sc, *, T, B, Hm, C):
    G = 4 * Hm

    lens = lens_ref[...]
    masks = [(lens > float(t)).astype(_F32) for t in range(T)]
    nmasks = [1.0 - mk for mk in masks]

    def cell(g, c):
        sg = jax.nn.sigmoid(g[:, 0:3 * Hm])
        gg = jnp.tanh(g[:, 3 * Hm:4 * Hm])
        c_n = sg[:, Hm:2 * Hm] * c + sg[:, 0:Hm] * gg
        h_n = sg[:, 2 * Hm:3 * Hm] * jnp.tanh(c_n)
        return h_n, c_n

    def step(s, gx, whh, st, collect):
        hf, cf, hb, cb = st
        t, tb = s, T - 1 - s
        hcat = jnp.concatenate([hf, hb], axis=-1).astype(_BF)
        g_rec = jnp.dot(hcat, whh, preferred_element_type=_F32)
        gf = gx[t * B:(t + 1) * B, 0:G] + g_rec[:, 0:G]
        gb = gx[tb * B:(tb + 1) * B, G:2 * G] + g_rec[:, G:2 * G]
        hf_n, cf_n = cell(gf, cf)
        hb_n, cb_n = cell(gb, cb)
        if collect:
            h1sc[t * B:(t + 1) * B, 0:Hm] = masks[t] * hf_n
            h1sc[tb * B:(tb + 1) * B, Hm:2 * Hm] = masks[tb] * hb_n
        hf = masks[t] * hf_n + nmasks[t] * hf
        cf = masks[t] * cf_n + nmasks[t] * cf
        hb = masks[tb] * hb_n + nmasks[tb] * hb
        cb = masks[tb] * cb_n + nmasks[tb] * cb
        return hf, cf, hb, cb

    whh1 = wbig[0, 0]
    wih2 = wbig[0, 1]
    whh2 = wbig[0, 2]

    x = x_ref[0].reshape(T * B, Hm)
    gx1 = jnp.dot(x, w00[0].astype(_BF), preferred_element_type=_F32) + w01[0]
    st = tuple(jnp.zeros((B, Hm), _F32) for _ in range(4))
    for s in range(T):
        st = step(s, gx1, whh1, st, True)
    h1f, h1b = st[0], st[2]

    h1 = h1sc[...]
    lnms = w05[0]
    mean = jnp.sum(h1 * lnms, axis=-1, keepdims=True)
    cen = h1 - mean
    var = jnp.sum(cen * cen * lnms, axis=-1, keepdims=True)
    normed = cen * lax.rsqrt(var + 1e-5) * w03[0] + w04[0]

    gx2 = jnp.dot(normed.astype(_BF), wih2, preferred_element_type=_F32) + w07[0]
    st = tuple(jnp.zeros((B, Hm), _F32) for _ in range(4))
    for s in range(T):
        st = step(s, gx2, whh2, st, False)
    h2f, h2b = st[0], st[2]

    feats = jnp.concatenate([h1f, h2f, h1b, h2b], axis=-1).astype(_BF)
    h = jnp.maximum(jnp.dot(feats, w09[0].astype(_BF),
                            preferred_element_type=_F32) + w10[0], 0.0)
    h = jnp.maximum(jnp.dot(h.astype(_BF), w11[0].astype(_BF),
                            preferred_element_type=_F32) + w12[0], 0.0)
    h = jnp.maximum(jnp.dot(h.astype(_BF), w13[0].astype(_BF),
                            preferred_element_type=_F32) + w14[0], 0.0)
    out_ref[0] = (jnp.dot(h.astype(_BF), w15[0].astype(_BF),
                          preferred_element_type=_F32) + w16[0]) * (1. / 3.)


def kernel(w00, w01, w02, w03, w04, w05, w06, w07, w08, w09, w10,
           w11, w12, w13, w14, w15, w16,
           embed, sentences, video, acoustic, lengths):
    Hm = w02.shape[1] // 2
    C = w15.shape[2]
    B, T = sentences.shape

    emb = embed[sentences]

    def prep(v):
        D = v.shape[2]
        x = jnp.transpose(v, (1, 0, 2)).astype(_BF)
        return jnp.pad(x, ((0, 0), (0, 0), (0, Hm - D)))

    x3 = jnp.stack([prep(emb), prep(video), prep(acoustic)])
    wbig = jnp.stack([w02, w06, w08], axis=1).astype(_BF)
    lens_col = lengths.astype(_F32).reshape(B, 1)

    weights = (w00, w01, w03, w04, w05, w07, w09, w10,
               w11, w12, w13, w14, w15, w16)

    kfn = functools.partial(_fused_kernel, T=T, B=B, Hm=Hm, C=C)

    in_specs = [pl.BlockSpec((B, 1), lambda m: (0, 0)),
                pl.BlockSpec((1, T, B, Hm), lambda m: (m, 0, 0, 0)),
                pl.BlockSpec((1, 3, 2 * Hm, 8 * Hm), lambda m: (m, 0, 0, 0))]
    in_specs += [pl.BlockSpec((1,) + w.shape[1:], lambda m: (m, 0, 0))
                 for w in weights]

    logits3 = pl.pallas_call(
        kfn,
        out_shape=jax.ShapeDtypeStruct((3, B, C), _F32),
        grid=(3,),
        in_specs=in_specs,
        out_specs=pl.BlockSpec((1, B, C), lambda m: (m, 0, 0)),
        scratch_shapes=[pltpu.VMEM((T * B, 2 * Hm), _F32)],
        compiler_params=pltpu.CompilerParams(
            dimension_semantics=("parallel",)),
    )(lens_col, x3, wbig, *weights)
    return jnp.sum(logits3, axis=0)


# speedup vs baseline: 1.7684x; 1.0039x over previous
import functools

import jax
import jax.numpy as jnp
from jax import lax
from jax.experimental import pallas as pl
from jax.experimental.pallas import tpu as pltpu

_BF = jnp.bfloat16
_F32 = jnp.float32


def _fused_kernel(lens_ref, x_ref, wbig,
                  w00, w01, w03, w04, w05, w07, w09, w10,
                  w11, w12, w13, w14, w15, w16,
                  out_ref, h1sc, *, T, B, Hm, C):
    G = 4 * Hm

    lens = lens_ref[...]
    masks = [(lens > float(t)).astype(_F32) for t in range(T)]
    nmasks = [1.0 - mk for mk in masks]

    def cell(g, c):
        sg = jax.nn.sigmoid(g[:, 0:3 * Hm])
        gg = jnp.tanh(g[:, 3 * Hm:4 * Hm])
        c_n = sg[:, Hm:2 * Hm] * c + sg[:, 0:Hm] * gg
        h_n = sg[:, 2 * Hm:3 * Hm] * jnp.tanh(c_n)
        return h_n, c_n

    def step(s, gx, whh, st, collect):
        hf, cf, hb, cb = st
        t, tb = s, T - 1 - s
        hcat = jnp.concatenate([hf, hb], axis=-1).astype(_BF)
        g_rec = jnp.dot(hcat, whh, preferred_element_type=_F32)
        gf = gx[t * B:(t + 1) * B, 0:G] + g_rec[:, 0:G]
        gb = gx[tb * B:(tb + 1) * B, G:2 * G] + g_rec[:, G:2 * G]
        hf_n, cf_n = cell(gf, cf)
        hb_n, cb_n = cell(gb, cb)
        if collect:
            h1sc[t * B:(t + 1) * B, 0:Hm] = masks[t] * hf_n
            h1sc[tb * B:(tb + 1) * B, Hm:2 * Hm] = masks[tb] * hb_n
        hf = masks[t] * hf_n + nmasks[t] * hf
        cf = masks[t] * cf_n + nmasks[t] * cf
        hb = masks[tb] * hb_n + nmasks[tb] * hb
        cb = masks[tb] * cb_n + nmasks[tb] * cb
        return hf, cf, hb, cb

    whh1 = wbig[0, 0]
    wih2 = wbig[0, 1]
    whh2 = wbig[0, 2]

    x = jnp.swapaxes(x_ref[0], 0, 1).reshape(T * B, Hm)
    gx1 = jnp.dot(x, w00[0].astype(_BF), preferred_element_type=_F32) + w01[0]
    st = tuple(jnp.zeros((B, Hm), _F32) for _ in range(4))
    for s in range(T):
        st = step(s, gx1, whh1, st, True)
    h1f, h1b = st[0], st[2]

    h1 = h1sc[...]
    lnms = w05[0]
    mean = jnp.sum(h1 * lnms, axis=-1, keepdims=True)
    cen = h1 - mean
    var = jnp.sum(cen * cen * lnms, axis=-1, keepdims=True)
    normed = cen * lax.rsqrt(var + 1e-5) * w03[0] + w04[0]

    gx2 = jnp.dot(normed.astype(_BF), wih2, preferred_element_type=_F32) + w07[0]
    st = tuple(jnp.zeros((B, Hm), _F32) for _ in range(4))
    for s in range(T):
        st = step(s, gx2, whh2, st, False)
    h2f, h2b = st[0], st[2]

    feats = jnp.concatenate([h1f, h2f, h1b, h2b], axis=-1).astype(_BF)
    h = jnp.maximum(jnp.dot(feats, w09[0].astype(_BF),
                            preferred_element_type=_F32) + w10[0], 0.0)
    h = jnp.maximum(jnp.dot(h.astype(_BF), w11[0].astype(_BF),
                            preferred_element_type=_F32) + w12[0], 0.0)
    h = jnp.maximum(jnp.dot(h.astype(_BF), w13[0].astype(_BF),
                            preferred_element_type=_F32) + w14[0], 0.0)
    out_ref[0] = (jnp.dot(h.astype(_BF), w15[0].astype(_BF),
                          preferred_element_type=_F32) + w16[0]) * (1. / 3.)


def kernel(w00, w01, w02, w03, w04, w05, w06, w07, w08, w09, w10,
           w11, w12, w13, w14, w15, w16,
           embed, sentences, video, acoustic, lengths):
    Hm = w02.shape[1] // 2
    C = w15.shape[2]
    B, T = sentences.shape

    emb = embed[sentences]

    def prep(v):
        D = v.shape[2]
        return jnp.pad(v.astype(_BF), ((0, 0), (0, 0), (0, Hm - D)))

    x3 = jnp.stack([prep(emb), prep(video), prep(acoustic)])
    wbig = jnp.stack([w02, w06, w08], axis=1).astype(_BF)
    lens_col = lengths.astype(_F32).reshape(B, 1)

    weights = (w00, w01, w03, w04, w05, w07, w09, w10,
               w11, w12, w13, w14, w15, w16)

    kfn = functools.partial(_fused_kernel, T=T, B=B, Hm=Hm, C=C)

    in_specs = [pl.BlockSpec((B, 1), lambda m: (0, 0)),
                pl.BlockSpec((1, B, T, Hm), lambda m: (m, 0, 0, 0)),
                pl.BlockSpec((1, 3, 2 * Hm, 8 * Hm), lambda m: (m, 0, 0, 0))]
    in_specs += [pl.BlockSpec((1,) + w.shape[1:], lambda m: (m, 0, 0))
                 for w in weights]

    logits3 = pl.pallas_call(
        kfn,
        out_shape=jax.ShapeDtypeStruct((3, B, C), _F32),
        grid=(3,),
        in_specs=in_specs,
        out_specs=pl.BlockSpec((1, B, C), lambda m: (m, 0, 0)),
        scratch_shapes=[pltpu.VMEM((T * B, 2 * Hm), _F32)],
        compiler_params=pltpu.CompilerParams(
            dimension_semantics=("parallel",)),
    )(lens_col, x3, wbig, *weights)
    return jnp.sum(logits3, axis=0)
